# Initial kernel scaffold; baseline (speedup 1.0000x reference)
import jax
import jax.numpy as jnp
from jax.experimental import pallas as pl


def kernel(hidden_states, top_k_index, top_k_weights, gate_up_proj, down_proj):
    raise NotImplementedError("write your pallas kernel here")


# speedup vs baseline: 1.1061x; 1.1061x over previous
import jax
import jax.numpy as jnp
from jax.experimental import pallas as pl


def _moe_body(x_ref, idx_ref, w_ref, gu_ref, dn_ref, out_ref):
    e = pl.program_id(0)
    inter = dn_ref.shape[-1]
    x = x_ref[...]
    wgu = gu_ref[0]
    gu = jax.lax.dot_general(
        x, wgu, (((1,), (1,)), ((), ())),
        preferred_element_type=jnp.float32)
    gate = gu[:, :inter]
    up = gu[:, inter:]
    h = gate * jax.nn.sigmoid(gate) * up
    wd = dn_ref[0]
    oe = jax.lax.dot_general(
        h, wd, (((1,), (1,)), ((), ())),
        preferred_element_type=jnp.float32)
    cw = jnp.sum(
        jnp.where(idx_ref[...] == e, w_ref[...], 0.0), axis=1)
    contrib = oe * cw[:, None]

    @pl.when(e == 0)
    def _init():
        out_ref[...] = contrib

    @pl.when(e != 0)
    def _acc():
        out_ref[...] += contrib


def kernel(hidden_states, top_k_index, top_k_weights, gate_up_proj, down_proj):
    T, H = hidden_states.shape
    E, I2, _ = gate_up_proj.shape
    I = down_proj.shape[-1]

    return pl.pallas_call(
        _moe_body,
        grid=(E,),
        in_specs=[
            pl.BlockSpec((T, H), lambda e: (0, 0)),
            pl.BlockSpec(top_k_index.shape, lambda e: (0, 0)),
            pl.BlockSpec(top_k_weights.shape, lambda e: (0, 0)),
            pl.BlockSpec((1, I2, H), lambda e: (e, 0, 0)),
            pl.BlockSpec((1, H, I), lambda e: (e, 0, 0)),
        ],
        out_specs=pl.BlockSpec((T, H), lambda e: (0, 0)),
        out_shape=jax.ShapeDtypeStruct((T, H), jnp.float32),
    )(hidden_states, top_k_index, top_k_weights, gate_up_proj, down_proj)


# speedup vs baseline: 1.1153x; 1.0083x over previous
import jax
import jax.numpy as jnp
from jax.experimental import pallas as pl


def _moe_body(x_ref, idx_ref, w_ref, wg_ref, wu_ref, dn_ref, out_ref):
    e = pl.program_id(0)
    x = x_ref[...]
    gate = jax.lax.dot_general(
        x, wg_ref[0], (((1,), (1,)), ((), ())),
        preferred_element_type=jnp.float32)
    up = jax.lax.dot_general(
        x, wu_ref[0], (((1,), (1,)), ((), ())),
        preferred_element_type=jnp.float32)
    h = gate * jax.nn.sigmoid(gate) * up
    wd = dn_ref[0]
    oe = jax.lax.dot_general(
        h, wd, (((1,), (1,)), ((), ())),
        preferred_element_type=jnp.float32)
    cw = jnp.sum(
        jnp.where(idx_ref[...] == e, w_ref[...], 0.0), axis=1)
    contrib = oe * cw[:, None]

    @pl.when(e == 0)
    def _init():
        out_ref[...] = contrib

    @pl.when(e != 0)
    def _acc():
        out_ref[...] += contrib


def kernel(hidden_states, top_k_index, top_k_weights, gate_up_proj, down_proj):
    T, H = hidden_states.shape
    E, I2, _ = gate_up_proj.shape
    I = down_proj.shape[-1]

    return pl.pallas_call(
        _moe_body,
        grid=(E,),
        in_specs=[
            pl.BlockSpec((T, H), lambda e: (0, 0)),
            pl.BlockSpec(top_k_index.shape, lambda e: (0, 0)),
            pl.BlockSpec(top_k_weights.shape, lambda e: (0, 0)),
            pl.BlockSpec((1, I, H), lambda e: (e, 0, 0)),
            pl.BlockSpec((1, I, H), lambda e: (e, 1, 0)),
            pl.BlockSpec((1, H, I), lambda e: (e, 0, 0)),
        ],
        out_specs=pl.BlockSpec((T, H), lambda e: (0, 0)),
        out_shape=jax.ShapeDtypeStruct((T, H), jnp.float32),
    )(hidden_states, top_k_index, top_k_weights,
      gate_up_proj, gate_up_proj, down_proj)
